# baseline (speedup 1.0000x reference)
import jax
import jax.numpy as jnp
from jax import lax
from jax.experimental import pallas as pl
from jax.experimental.pallas import tpu as pltpu

_EPS = 1e-5
_INV_SQRT2 = 0.7071067811865475


def _one_image(x, w3_ref, w5_ref, wp_ref, bias_ref, gamma_ref, beta_ref):
    C, HW = x.shape
    W = 32

    lane = lax.broadcasted_iota(jnp.int32, (1, HW), 1)
    w_idx = lane & (W - 1)

    def broll(v, shift):
        vi = pltpu.bitcast(v, jnp.int32)
        return pltpu.bitcast(pltpu.roll(vi, shift % HW, axis=1), jnp.bfloat16)

    xb = x
    xw = {0: xb}
    for dw in (-2, -1, 1, 2):
        wv = w_idx + dw
        maskf = jnp.where((wv >= 0) & (wv < W), 1.0, 0.0).astype(jnp.bfloat16)
        xw[dw] = broll(xb, -dw) * maskf

    def banded(taps, wt_ref, r):
        acc = None
        for dw in range(-r, r + 1):
            t = taps
            coef = wt_ref[:, t + dw + r:t + dw + r + 1]
            term = xw[dw] * coef
            acc = term if acc is None else acc + term
        return acc

    def dwconv(wt_ref, r):
        n = 2 * r + 1
        f = banded(r * n, wt_ref, r)
        for dh in (d for d in range(-r, r + 1) if d != 0):
            p = banded((dh + r) * n, wt_ref, r)
            hv = (lane >> 5) + dh
            hmask = jnp.where((hv >= 0) & (hv < W), 1.0, 0.0
                              ).astype(jnp.bfloat16)
            f = f + broll(p, -dh * W) * hmask
        return f

    def tab_dot(lhs, rhs):
        return lax.dot_general(lhs, rhs, (((0,), (1,)), ((), ())),
                               preferred_element_type=jnp.float32)

    acc = tab_dot(xb, wp_ref[:, 0:C])
    f3 = dwconv(w3_ref, 1)
    acc = acc + tab_dot(f3, wp_ref[:, C:2 * C])
    f5 = dwconv(w5_ref, 2)
    acc = acc + tab_dot(f5, wp_ref[:, 2 * C:3 * C])
    acc = acc + bias_ref[...]

    mu = jnp.mean(acc, axis=1, keepdims=True)
    xc = acc - mu
    var = jnp.mean(xc * xc, axis=1, keepdims=True)
    y = xc * lax.rsqrt(var + _EPS)
    y = y * gamma_ref[...] + beta_ref[...]
    return 0.5 * y * (1.0 + lax.erf(y * _INV_SQRT2))


def _lmf_kernel(x_ref, w3_ref, w5_ref, wp_ref, bias_ref, gamma_ref, beta_ref,
                o_ref):
    o_ref[0] = _one_image(x_ref[0], w3_ref, w5_ref, wp_ref, bias_ref,
                          gamma_ref, beta_ref)


def kernel(x_in, w3, b3, w5, b5, wp, bp, gamma, beta):
    B, C, H, W = x_in.shape
    D = wp.shape[0]
    HW = H * W

    x2 = x_in.astype(jnp.bfloat16).reshape(B, C, HW)
    w3f = w3.reshape(C, 9).astype(jnp.bfloat16)
    w5f = w5.reshape(C, 25).astype(jnp.bfloat16)
    wpb = wp.astype(jnp.bfloat16)
    bias = (bp + wp[:, C:2 * C] @ b3 + wp[:, 2 * C:] @ b5).reshape(1, D)

    grid = (B,)
    out = pl.pallas_call(
        _lmf_kernel,
        grid=grid,
        in_specs=[
            pl.BlockSpec((1, C, HW), lambda b: (b, 0, 0)),
            pl.BlockSpec((C, 9), lambda b: (0, 0)),
            pl.BlockSpec((C, 25), lambda b: (0, 0)),
            pl.BlockSpec((D, 3 * C), lambda b: (0, 0)),
            pl.BlockSpec((1, D), lambda b: (0, 0)),
            pl.BlockSpec((1, D), lambda b: (0, 0)),
            pl.BlockSpec((1, D), lambda b: (0, 0)),
        ],
        out_specs=pl.BlockSpec((1, HW, D), lambda b: (b, 0, 0)),
        out_shape=jax.ShapeDtypeStruct((B, HW, D), jnp.float32),
        compiler_params=pltpu.CompilerParams(
            dimension_semantics=("parallel",),
            vmem_limit_bytes=100 * 1024 * 1024,
        ),
    )(x2, w3f, w5f, wpb, bias, gamma.reshape(1, D), beta.reshape(1, D))
    return out


# speedup vs baseline: 1.2833x; 1.2833x over previous
import jax
import jax.numpy as jnp
from jax import lax
from jax.experimental import pallas as pl
from jax.experimental.pallas import tpu as pltpu

_EPS = 1e-5
_INV_SQRT2 = 0.7071067811865475


def _one_image(x, w3_ref, w5_ref, wp_ref, bias_ref, gamma_ref, beta_ref):
    C, HW = x.shape
    W = 32

    lane = lax.broadcasted_iota(jnp.int32, (1, HW), 1)
    w_idx = lane & (W - 1)

    def broll(v, shift):
        vi = pltpu.bitcast(v, jnp.int32)
        return pltpu.bitcast(pltpu.roll(vi, shift % HW, axis=1), jnp.bfloat16)

    xb = x.astype(jnp.bfloat16)
    xw = {0: xb}
    for dw in (-2, -1, 1, 2):
        wv = w_idx + dw
        maskf = jnp.where((wv >= 0) & (wv < W), 1.0, 0.0).astype(jnp.bfloat16)
        xw[dw] = broll(xb, -dw) * maskf

    def banded(taps, wt_ref, r):
        acc = None
        for dw in range(-r, r + 1):
            t = taps + dw + r
            coef = pltpu.repeat(wt_ref[:, t * 128:(t + 1) * 128], HW // 128,
                                axis=1)
            term = xw[dw] * coef
            acc = term if acc is None else acc + term
        return acc

    def dwconv(wt_ref, r):
        n = 2 * r + 1
        f = banded(r * n, wt_ref, r)
        for dh in (d for d in range(-r, r + 1) if d != 0):
            p = banded((dh + r) * n, wt_ref, r)
            hv = (lane >> 5) + dh
            hmask = jnp.where((hv >= 0) & (hv < W), 1.0, 0.0
                              ).astype(jnp.bfloat16)
            f = f + broll(p, -dh * W) * hmask
        return f

    def tab_dot(lhs, rhs):
        return lax.dot_general(lhs, rhs, (((0,), (1,)), ((), ())),
                               preferred_element_type=jnp.float32)

    acc = tab_dot(xb, wp_ref[:, 0:C])
    f3 = dwconv(w3_ref, 1)
    acc = acc + tab_dot(f3, wp_ref[:, C:2 * C])
    f5 = dwconv(w5_ref, 2)
    acc = acc + tab_dot(f5, wp_ref[:, 2 * C:3 * C])
    acc = acc + bias_ref[...]

    mu = jnp.mean(acc, axis=1, keepdims=True)
    xc = acc - mu
    var = jnp.mean(xc * xc, axis=1, keepdims=True)
    y = xc * lax.rsqrt(var + _EPS)
    y = y * gamma_ref[...] + beta_ref[...]
    return 0.5 * y * (1.0 + lax.erf(y * _INV_SQRT2))


def _lmf_kernel(x_ref, w3_ref, w5_ref, wp_ref, bias_ref, gamma_ref, beta_ref,
                o_ref):
    o_ref[0] = _one_image(x_ref[0], w3_ref, w5_ref, wp_ref, bias_ref,
                          gamma_ref, beta_ref)


def kernel(x_in, w3, b3, w5, b5, wp, bp, gamma, beta):
    B, C, H, W = x_in.shape
    D = wp.shape[0]
    HW = H * W

    x2 = x_in.reshape(B, C, HW)
    w3f = jnp.broadcast_to(w3.reshape(C, 9, 1), (C, 9, 128)
                           ).reshape(C, 9 * 128).astype(jnp.bfloat16)
    w5f = jnp.broadcast_to(w5.reshape(C, 25, 1), (C, 25, 128)
                           ).reshape(C, 25 * 128).astype(jnp.bfloat16)
    wpb = wp.astype(jnp.bfloat16)
    bias = (bp + wp[:, C:2 * C] @ b3 + wp[:, 2 * C:] @ b5).reshape(1, D)

    grid = (B,)
    out = pl.pallas_call(
        _lmf_kernel,
        grid=grid,
        in_specs=[
            pl.BlockSpec((1, C, HW), lambda b: (b, 0, 0)),
            pl.BlockSpec((C, 9 * 128), lambda b: (0, 0)),
            pl.BlockSpec((C, 25 * 128), lambda b: (0, 0)),
            pl.BlockSpec((D, 3 * C), lambda b: (0, 0)),
            pl.BlockSpec((1, D), lambda b: (0, 0)),
            pl.BlockSpec((1, D), lambda b: (0, 0)),
            pl.BlockSpec((1, D), lambda b: (0, 0)),
        ],
        out_specs=pl.BlockSpec((1, HW, D), lambda b: (b, 0, 0)),
        out_shape=jax.ShapeDtypeStruct((B, HW, D), jnp.float32),
        compiler_params=pltpu.CompilerParams(
            dimension_semantics=("parallel",),
            vmem_limit_bytes=100 * 1024 * 1024,
        ),
    )(x2, w3f, w5f, wpb, bias, gamma.reshape(1, D), beta.reshape(1, D))
    return out
